# Initial kernel scaffold; baseline (speedup 1.0000x reference)
import jax
import jax.numpy as jnp
from jax.experimental import pallas as pl


def kernel(x, g1, Wq, Wk, Wv, Wo, g2, Wr, W1, W2, W3):
    raise NotImplementedError("write your pallas kernel here")


# speedup vs baseline: 1.1600x; 1.1600x over previous
import functools
import math

import jax
import jax.numpy as jnp
from jax.experimental import pallas as pl
from jax.experimental.pallas import tpu as pltpu

HIDDEN = 768
FFN = 2048
HEADS = 12
DH = 64
HALF = 32
E = 8
CTX = 2048
BS = 256
QB = 256
BLK = 256
NBMAX = (CTX * 2 + E * (BLK - 1) + BLK - 1) // BLK
F32 = jnp.float32
BF16 = jnp.bfloat16


def _k1_body(x_ref, g1_ref, wq_ref, wk_ref, wv_ref, cos_ref, sin_ref,
             q_ref, k_ref, v_ref):
    xb = x_ref[...]
    z = xb * jax.lax.rsqrt(jnp.mean(xb * xb, axis=-1, keepdims=True) + 1e-5)
    z = (z * g1_ref[...]).astype(BF16)
    qf = jnp.dot(z, wq_ref[...], preferred_element_type=F32)
    kf = jnp.dot(z, wk_ref[...], preferred_element_type=F32)
    vf = jnp.dot(z, wv_ref[...], preferred_element_type=F32)
    c = cos_ref[...]
    s = sin_ref[...]
    for h in range(HEADS):
        a = slice(DH * h, DH * h + HALF)
        b = slice(DH * h + HALF, DH * h + DH)
        q1, q2 = qf[:, a], qf[:, b]
        q_ref[h, :, 0:HALF] = (q1 * c - q2 * s).astype(BF16)
        q_ref[h, :, HALF:DH] = (q1 * s + q2 * c).astype(BF16)
        k1_, k2_ = kf[:, a], kf[:, b]
        k_ref[h, :, 0:HALF] = (k1_ * c - k2_ * s).astype(BF16)
        k_ref[h, :, HALF:DH] = (k1_ * s + k2_ * c).astype(BF16)
        v_ref[h, :, :] = vf[:, DH * h:DH * h + DH].astype(BF16)


def _qkv_rope(xs, g1r, wq, wk, wv, cos_t, sin_t):
    shp = jax.ShapeDtypeStruct((HEADS, CTX, DH), BF16)
    return pl.pallas_call(
        _k1_body,
        grid=(CTX // BS,),
        in_specs=[
            pl.BlockSpec((BS, HIDDEN), lambda i: (i, 0)),
            pl.BlockSpec((1, HIDDEN), lambda i: (0, 0)),
            pl.BlockSpec((HIDDEN, HIDDEN), lambda i: (0, 0)),
            pl.BlockSpec((HIDDEN, HIDDEN), lambda i: (0, 0)),
            pl.BlockSpec((HIDDEN, HIDDEN), lambda i: (0, 0)),
            pl.BlockSpec((BS, HALF), lambda i: (i, 0)),
            pl.BlockSpec((BS, HALF), lambda i: (i, 0)),
        ],
        out_specs=[pl.BlockSpec((HEADS, BS, DH), lambda i: (0, i, 0))] * 3,
        out_shape=[shp, shp, shp],
    )(xs, g1r, wq, wk, wv, cos_t, sin_t)


def _k2_body(q_ref, k_ref, v_ref, o_ref):
    i = pl.program_id(1)
    q = q_ref[0]
    scale = 1.0 / math.sqrt(DH)
    rows = jax.lax.broadcasted_iota(jnp.int32, (QB, QB), 0) + i * QB

    def body(j, carry):
        m, l, acc = carry
        kb = k_ref[0, pl.ds(j * QB, QB), :]
        vb = v_ref[0, pl.ds(j * QB, QB), :]
        s = jax.lax.dot_general(q, kb, (((1,), (1,)), ((), ())),
                                preferred_element_type=F32) * scale
        cols = jax.lax.broadcasted_iota(jnp.int32, (QB, QB), 1) + j * QB
        s = jnp.where(rows >= cols, s, -1e30)
        m_new = jnp.maximum(m, jnp.max(s, axis=-1, keepdims=True))
        p = jnp.exp(s - m_new)
        corr = jnp.exp(m - m_new)
        l_new = l * corr + jnp.sum(p, axis=-1, keepdims=True)
        acc_new = acc * corr + jnp.dot(p.astype(BF16), vb,
                                       preferred_element_type=F32)
        return m_new, l_new, acc_new

    m0 = jnp.full((QB, 1), -1e30, F32)
    l0 = jnp.zeros((QB, 1), F32)
    a0 = jnp.zeros((QB, DH), F32)
    m, l, acc = jax.lax.fori_loop(0, i + 1, body, (m0, l0, a0))
    o_ref[0] = (acc / l).astype(BF16)


def _flash(q, k, v):
    return pl.pallas_call(
        _k2_body,
        grid=(HEADS, CTX // QB),
        in_specs=[
            pl.BlockSpec((1, QB, DH), lambda h, i: (h, i, 0)),
            pl.BlockSpec((1, CTX, DH), lambda h, i: (h, 0, 0)),
            pl.BlockSpec((1, CTX, DH), lambda h, i: (h, 0, 0)),
        ],
        out_specs=pl.BlockSpec((1, QB, DH), lambda h, i: (h, i, 0)),
        out_shape=jax.ShapeDtypeStruct((HEADS, CTX, DH), BF16),
    )(q, k, v)


def _k3_body(o_ref, x_ref, wo_ref, g2_ref, wr_ref, h_ref, z2_ref, lg_ref):
    oc = jnp.concatenate([o_ref[h] for h in range(HEADS)], axis=-1)
    attn = jnp.dot(oc, wo_ref[...], preferred_element_type=F32)
    hb = x_ref[...] + attn
    h_ref[...] = hb
    z2 = hb * jax.lax.rsqrt(jnp.mean(hb * hb, axis=-1, keepdims=True) + 1e-5)
    z2 = z2 * g2_ref[...]
    z2_ref[...] = z2.astype(BF16)
    lg_ref[...] = jnp.dot(z2, wr_ref[...], preferred_element_type=F32)


def _postattn(o, xs, wo, g2r, wr):
    return pl.pallas_call(
        _k3_body,
        grid=(CTX // BS,),
        in_specs=[
            pl.BlockSpec((HEADS, BS, DH), lambda i: (0, i, 0)),
            pl.BlockSpec((BS, HIDDEN), lambda i: (i, 0)),
            pl.BlockSpec((HIDDEN, HIDDEN), lambda i: (0, 0)),
            pl.BlockSpec((1, HIDDEN), lambda i: (0, 0)),
            pl.BlockSpec((HIDDEN, E), lambda i: (0, 0)),
        ],
        out_specs=[
            pl.BlockSpec((BS, HIDDEN), lambda i: (i, 0)),
            pl.BlockSpec((BS, HIDDEN), lambda i: (i, 0)),
            pl.BlockSpec((BS, E), lambda i: (i, 0)),
        ],
        out_shape=[
            jax.ShapeDtypeStruct((CTX, HIDDEN), F32),
            jax.ShapeDtypeStruct((CTX, HIDDEN), BF16),
            jax.ShapeDtypeStruct((CTX, E), F32),
        ],
    )(o, xs, wo, g2r, wr)


def _k4_body(lg_ref, pos0_ref, pos1_ref, g0_ref, g1_ref, noff_ref, aux_ref):
    lg = lg_ref[...]
    mx = jnp.max(lg, axis=-1, keepdims=True)
    ex = jnp.exp(lg - mx)
    probs = ex / jnp.sum(ex, axis=-1, keepdims=True)
    e_iota = jax.lax.broadcasted_iota(jnp.int32, (CTX, E), 1).astype(F32)
    m1 = jnp.max(probs, axis=-1, keepdims=True)
    i1 = jnp.min(jnp.where(probs == m1, e_iota, float(E)), axis=-1,
                 keepdims=True)
    probs_m = jnp.where(e_iota == i1, -1.0, probs)
    m2 = jnp.max(probs_m, axis=-1, keepdims=True)
    i2 = jnp.min(jnp.where(probs_m == m2, e_iota, float(E)), axis=-1,
                 keepdims=True)
    den = m1 + m2
    g0_ref[...] = m1 / den
    g1_ref[...] = m2 / den
    sel0 = (e_iota == i1)
    sel1 = (e_iota == i2)
    oh = sel0.astype(F32) + sel1.astype(F32)

    ri = jax.lax.broadcasted_iota(jnp.int32, (BS, BS), 0)
    ci = jax.lax.broadcasted_iota(jnp.int32, (BS, BS), 1)
    tri = (ci < ri).astype(BF16)
    carry = jnp.zeros((1, E), F32)
    parts = []
    for b in range(CTX // BS):
        seg = oh[b * BS:(b + 1) * BS, :]
        parts.append(jnp.dot(tri, seg.astype(BF16),
                             preferred_element_type=F32) + carry)
        carry = carry + jnp.sum(seg, axis=0, keepdims=True)
    c = jnp.concatenate(parts, axis=0)
    n = carry
    padded = jnp.floor((n + (BLK - 1)) / BLK) * BLK
    offs = []
    run = jnp.zeros((1, 1), F32)
    for e in range(E):
        offs.append(run)
        run = run + padded[:, e:e + 1]
    off = jnp.concatenate(offs, axis=1)
    c2 = c + off
    pos0_ref[...] = jnp.sum(jnp.where(sel0, c2, 0.0), axis=-1, keepdims=True)
    pos1_ref[...] = jnp.sum(jnp.where(sel1, c2, 0.0), axis=-1, keepdims=True)
    noff_ref[...] = jnp.concatenate([n, off], axis=0)
    f = jnp.mean(oh, axis=0, keepdims=True)
    pbar = jnp.mean(probs, axis=0, keepdims=True)
    aux_ref[...] = float(E) * jnp.sum(f * pbar, axis=-1, keepdims=True)


def _route(lg):
    return pl.pallas_call(
        _k4_body,
        out_shape=[
            jax.ShapeDtypeStruct((CTX, 1), F32),
            jax.ShapeDtypeStruct((CTX, 1), F32),
            jax.ShapeDtypeStruct((CTX, 1), F32),
            jax.ShapeDtypeStruct((CTX, 1), F32),
            jax.ShapeDtypeStruct((2, E), F32),
            jax.ShapeDtypeStruct((1, 1), F32),
        ],
    )(lg)


def _k5_body(be_ref, valid_ref, z2_ref, p0t_ref, p1t_ref, p0_ref, p1_ref,
             g0_ref, g1_ref, h_ref, w1_ref, w3_ref, w2_ref, out_ref, acc_ref):
    b = pl.program_id(0)

    @pl.when(b == 0)
    def _():
        acc_ref[...] = jnp.zeros_like(acc_ref)

    @pl.when(valid_ref[b] == 1)
    def _():
        jbase = (b * BLK).astype(F32)
        sub = jax.lax.broadcasted_iota(jnp.int32, (BLK, CTX), 0).astype(F32) + jbase
        dt = jnp.logical_or(p0t_ref[...] == sub, p1t_ref[...] == sub)
        zs = jnp.dot(dt.astype(BF16), z2_ref[...], preferred_element_type=F32)
        zsb = zs.astype(BF16)
        h1 = jnp.dot(zsb, w1_ref[0], preferred_element_type=F32)
        h3 = jnp.dot(zsb, w3_ref[0], preferred_element_type=F32)
        hh = (h1 * jax.nn.sigmoid(h1) * h3).astype(BF16)
        eo = jnp.dot(hh, w2_ref[0], preferred_element_type=F32)
        lane = jax.lax.broadcasted_iota(jnp.int32, (CTX, BLK), 1).astype(F32) + jbase
        comb = (g0_ref[...] * (p0_ref[...] == lane) +
                g1_ref[...] * (p1_ref[...] == lane))
        acc_ref[...] += jnp.dot(comb.astype(BF16), eo.astype(BF16),
                                preferred_element_type=F32)

    @pl.when(b == NBMAX - 1)
    def _():
        out_ref[...] = acc_ref[...] + h_ref[...]


def _moe(be, valid, z2b, p0t, p1t, p0, p1, g0, g1, hs, w1, w3, w2):
    grid_spec = pltpu.PrefetchScalarGridSpec(
        num_scalar_prefetch=2,
        grid=(NBMAX,),
        in_specs=[
            pl.BlockSpec((CTX, HIDDEN), lambda b, be, vl: (0, 0)),
            pl.BlockSpec((1, CTX), lambda b, be, vl: (0, 0)),
            pl.BlockSpec((1, CTX), lambda b, be, vl: (0, 0)),
            pl.BlockSpec((CTX, 1), lambda b, be, vl: (0, 0)),
            pl.BlockSpec((CTX, 1), lambda b, be, vl: (0, 0)),
            pl.BlockSpec((CTX, 1), lambda b, be, vl: (0, 0)),
            pl.BlockSpec((CTX, 1), lambda b, be, vl: (0, 0)),
            pl.BlockSpec((CTX, HIDDEN), lambda b, be, vl: (0, 0)),
            pl.BlockSpec((1, HIDDEN, FFN), lambda b, be, vl: (be[b], 0, 0)),
            pl.BlockSpec((1, HIDDEN, FFN), lambda b, be, vl: (be[b], 0, 0)),
            pl.BlockSpec((1, FFN, HIDDEN), lambda b, be, vl: (be[b], 0, 0)),
        ],
        out_specs=pl.BlockSpec((CTX, HIDDEN), lambda b, be, vl: (0, 0)),
        scratch_shapes=[pltpu.VMEM((CTX, HIDDEN), F32)],
    )
    return pl.pallas_call(
        _k5_body,
        grid_spec=grid_spec,
        out_shape=jax.ShapeDtypeStruct((CTX, HIDDEN), F32),
    )(be, valid, z2b, p0t, p1t, p0, p1, g0, g1, hs, w1, w3, w2)


def kernel(x, g1, Wq, Wk, Wv, Wo, g2, Wr, W1, W2, W3):
    xs = x.reshape(CTX, HIDDEN)
    g1r = g1.reshape(1, HIDDEN)
    g2r = g2.reshape(1, HIDDEN)
    freqs = 1.0 / (10000.0 ** (jnp.arange(HALF, dtype=F32) / HALF))
    t = jnp.arange(CTX, dtype=F32)
    ang = t[:, None] * freqs[None, :]
    cos_t = jnp.cos(ang)
    sin_t = jnp.sin(ang)

    q, k, v = _qkv_rope(xs, g1r, Wq.astype(BF16), Wk.astype(BF16),
                        Wv.astype(BF16), cos_t, sin_t)
    o = _flash(q, k, v)
    hs, z2b, lg = _postattn(o, xs, Wo.astype(BF16), g2r, Wr)
    p0, p1, g0v, g1v, noff, aux = _route(lg)

    n = noff[0]
    off = noff[1]
    pad = jnp.floor((n + (BLK - 1)) / BLK) * BLK
    bidx = (jnp.arange(NBMAX, dtype=F32) * BLK)[:, None]
    inseg = (bidx >= off[None, :]) & (bidx < (off + pad)[None, :])
    be = jnp.sum(inseg * jnp.arange(E, dtype=F32)[None, :], axis=1)
    valid = bidx[:, 0] < jnp.sum(pad)
    be = jnp.where(valid, be, float(E - 1)).astype(jnp.int32)
    valid = valid.astype(jnp.int32)

    h2 = _moe(be, valid, z2b, p0.reshape(1, CTX), p1.reshape(1, CTX),
              p0, p1, g0v, g1v, hs,
              W1.astype(BF16), W3.astype(BF16), W2.astype(BF16))
    return h2.reshape(1, CTX, HIDDEN), aux.reshape(())


# speedup vs baseline: 1.1670x; 1.0061x over previous
import functools
import math

import jax
import jax.numpy as jnp
from jax.experimental import pallas as pl
from jax.experimental.pallas import tpu as pltpu

HIDDEN = 768
FFN = 2048
HEADS = 12
DH = 64
HALF = 32
E = 8
CTX = 2048
BS = 256
QB = 256
BLK = 256
NBMAX = (CTX * 2 + E * (BLK - 1) + BLK - 1) // BLK
F32 = jnp.float32
BF16 = jnp.bfloat16


def _k1_body(x_ref, g1_ref, wq_ref, wk_ref, wv_ref, cos_ref, sin_ref,
             q_ref, k_ref, v_ref):
    xb = x_ref[...]
    z = xb * jax.lax.rsqrt(jnp.mean(xb * xb, axis=-1, keepdims=True) + 1e-5)
    z = (z * g1_ref[...]).astype(BF16)
    qf = jnp.dot(z, wq_ref[...], preferred_element_type=F32)
    kf = jnp.dot(z, wk_ref[...], preferred_element_type=F32)
    vf = jnp.dot(z, wv_ref[...], preferred_element_type=F32)
    c = cos_ref[...]
    s = sin_ref[...]
    for h in range(HEADS):
        a = slice(DH * h, DH * h + HALF)
        b = slice(DH * h + HALF, DH * h + DH)
        q1, q2 = qf[:, a], qf[:, b]
        q_ref[h, :, 0:HALF] = (q1 * c - q2 * s).astype(BF16)
        q_ref[h, :, HALF:DH] = (q1 * s + q2 * c).astype(BF16)
        k1_, k2_ = kf[:, a], kf[:, b]
        k_ref[h, :, 0:HALF] = (k1_ * c - k2_ * s).astype(BF16)
        k_ref[h, :, HALF:DH] = (k1_ * s + k2_ * c).astype(BF16)
        v_ref[h, :, :] = vf[:, DH * h:DH * h + DH].astype(BF16)


def _qkv_rope(xs, g1r, wq, wk, wv, cos_t, sin_t):
    shp = jax.ShapeDtypeStruct((HEADS, CTX, DH), BF16)
    return pl.pallas_call(
        _k1_body,
        grid=(CTX // BS,),
        in_specs=[
            pl.BlockSpec((BS, HIDDEN), lambda i: (i, 0)),
            pl.BlockSpec((1, HIDDEN), lambda i: (0, 0)),
            pl.BlockSpec((HIDDEN, HIDDEN), lambda i: (0, 0)),
            pl.BlockSpec((HIDDEN, HIDDEN), lambda i: (0, 0)),
            pl.BlockSpec((HIDDEN, HIDDEN), lambda i: (0, 0)),
            pl.BlockSpec((BS, HALF), lambda i: (i, 0)),
            pl.BlockSpec((BS, HALF), lambda i: (i, 0)),
        ],
        out_specs=[pl.BlockSpec((HEADS, BS, DH), lambda i: (0, i, 0))] * 3,
        out_shape=[shp, shp, shp],
    )(xs, g1r, wq, wk, wv, cos_t, sin_t)


def _k2_body(q_ref, k_ref, v_ref, o_ref):
    i = pl.program_id(1)
    q = q_ref[0]
    scale = 1.0 / math.sqrt(DH)
    rows = jax.lax.broadcasted_iota(jnp.int32, (QB, QB), 0) + i * QB

    def body(j, carry):
        m, l, acc = carry
        kb = k_ref[0, pl.ds(j * QB, QB), :]
        vb = v_ref[0, pl.ds(j * QB, QB), :]
        s = jax.lax.dot_general(q, kb, (((1,), (1,)), ((), ())),
                                preferred_element_type=F32) * scale
        cols = jax.lax.broadcasted_iota(jnp.int32, (QB, QB), 1) + j * QB
        s = jnp.where(rows >= cols, s, -1e30)
        m_new = jnp.maximum(m, jnp.max(s, axis=-1, keepdims=True))
        p = jnp.exp(s - m_new)
        corr = jnp.exp(m - m_new)
        l_new = l * corr + jnp.sum(p, axis=-1, keepdims=True)
        acc_new = acc * corr + jnp.dot(p.astype(BF16), vb,
                                       preferred_element_type=F32)
        return m_new, l_new, acc_new

    m0 = jnp.full((QB, 1), -1e30, F32)
    l0 = jnp.zeros((QB, 1), F32)
    a0 = jnp.zeros((QB, DH), F32)
    m, l, acc = jax.lax.fori_loop(0, i + 1, body, (m0, l0, a0))
    o_ref[0] = (acc / l).astype(BF16)


def _flash(q, k, v):
    return pl.pallas_call(
        _k2_body,
        grid=(HEADS, CTX // QB),
        in_specs=[
            pl.BlockSpec((1, QB, DH), lambda h, i: (h, i, 0)),
            pl.BlockSpec((1, CTX, DH), lambda h, i: (h, 0, 0)),
            pl.BlockSpec((1, CTX, DH), lambda h, i: (h, 0, 0)),
        ],
        out_specs=pl.BlockSpec((1, QB, DH), lambda h, i: (h, i, 0)),
        out_shape=jax.ShapeDtypeStruct((HEADS, CTX, DH), BF16),
    )(q, k, v)


def _k3_body(o_ref, x_ref, wo_ref, g2_ref, wr_ref, h_ref, z2_ref, lg_ref):
    oc = jnp.concatenate([o_ref[h] for h in range(HEADS)], axis=-1)
    attn = jnp.dot(oc, wo_ref[...], preferred_element_type=F32)
    hb = x_ref[...] + attn
    h_ref[...] = hb
    z2 = hb * jax.lax.rsqrt(jnp.mean(hb * hb, axis=-1, keepdims=True) + 1e-5)
    z2 = z2 * g2_ref[...]
    z2_ref[...] = z2.astype(BF16)
    lg_ref[...] = jnp.dot(z2, wr_ref[...], preferred_element_type=F32)


def _postattn(o, xs, wo, g2r, wr):
    return pl.pallas_call(
        _k3_body,
        grid=(CTX // BS,),
        in_specs=[
            pl.BlockSpec((HEADS, BS, DH), lambda i: (0, i, 0)),
            pl.BlockSpec((BS, HIDDEN), lambda i: (i, 0)),
            pl.BlockSpec((HIDDEN, HIDDEN), lambda i: (0, 0)),
            pl.BlockSpec((1, HIDDEN), lambda i: (0, 0)),
            pl.BlockSpec((HIDDEN, E), lambda i: (0, 0)),
        ],
        out_specs=[
            pl.BlockSpec((BS, HIDDEN), lambda i: (i, 0)),
            pl.BlockSpec((BS, HIDDEN), lambda i: (i, 0)),
            pl.BlockSpec((BS, E), lambda i: (i, 0)),
        ],
        out_shape=[
            jax.ShapeDtypeStruct((CTX, HIDDEN), F32),
            jax.ShapeDtypeStruct((CTX, HIDDEN), BF16),
            jax.ShapeDtypeStruct((CTX, E), F32),
        ],
    )(o, xs, wo, g2r, wr)


def _k4_body(lg_ref, pos0_ref, pos1_ref, g0_ref, g1_ref, noff_ref, aux_ref):
    lg = lg_ref[...]
    mx = jnp.max(lg, axis=-1, keepdims=True)
    ex = jnp.exp(lg - mx)
    probs = ex / jnp.sum(ex, axis=-1, keepdims=True)
    e_iota = jax.lax.broadcasted_iota(jnp.int32, (CTX, E), 1).astype(F32)
    m1 = jnp.max(probs, axis=-1, keepdims=True)
    i1 = jnp.min(jnp.where(probs == m1, e_iota, float(E)), axis=-1,
                 keepdims=True)
    probs_m = jnp.where(e_iota == i1, -1.0, probs)
    m2 = jnp.max(probs_m, axis=-1, keepdims=True)
    i2 = jnp.min(jnp.where(probs_m == m2, e_iota, float(E)), axis=-1,
                 keepdims=True)
    den = m1 + m2
    g0_ref[...] = m1 / den
    g1_ref[...] = m2 / den
    sel0 = (e_iota == i1)
    sel1 = (e_iota == i2)
    oh = sel0.astype(F32) + sel1.astype(F32)

    ri = jax.lax.broadcasted_iota(jnp.int32, (BS, BS), 0)
    ci = jax.lax.broadcasted_iota(jnp.int32, (BS, BS), 1)
    tri = (ci < ri).astype(BF16)
    carry = jnp.zeros((1, E), F32)
    parts = []
    for b in range(CTX // BS):
        seg = oh[b * BS:(b + 1) * BS, :]
        parts.append(jnp.dot(tri, seg.astype(BF16),
                             preferred_element_type=F32) + carry)
        carry = carry + jnp.sum(seg, axis=0, keepdims=True)
    c = jnp.concatenate(parts, axis=0)
    n = carry
    padded = jnp.floor((n + (BLK - 1)) / BLK) * BLK
    offs = []
    run = jnp.zeros((1, 1), F32)
    for e in range(E):
        offs.append(run)
        run = run + padded[:, e:e + 1]
    off = jnp.concatenate(offs, axis=1)
    c2 = c + off
    pos0_ref[...] = jnp.sum(jnp.where(sel0, c2, 0.0), axis=-1, keepdims=True)
    pos1_ref[...] = jnp.sum(jnp.where(sel1, c2, 0.0), axis=-1, keepdims=True)
    noff_ref[...] = jnp.concatenate([n, off], axis=0)
    f = jnp.mean(oh, axis=0, keepdims=True)
    pbar = jnp.mean(probs, axis=0, keepdims=True)
    aux_ref[...] = float(E) * jnp.sum(f * pbar, axis=-1, keepdims=True)


def _route(lg):
    return pl.pallas_call(
        _k4_body,
        out_shape=[
            jax.ShapeDtypeStruct((CTX, 1), F32),
            jax.ShapeDtypeStruct((CTX, 1), F32),
            jax.ShapeDtypeStruct((CTX, 1), F32),
            jax.ShapeDtypeStruct((CTX, 1), F32),
            jax.ShapeDtypeStruct((2, E), F32),
            jax.ShapeDtypeStruct((1, 1), F32),
        ],
    )(lg)


FC = FFN // 2
NFC = FFN // FC


def _k5_body(be_ref, valid_ref, z2_ref, p0t_ref, p1t_ref, p0_ref, p1_ref,
             g0_ref, g1_ref, h_ref, w1_ref, w3_ref, w2_ref, out_ref,
             acc_ref, zs_ref, eo_ref):
    b = pl.program_id(0)
    fc = pl.program_id(1)

    @pl.when(jnp.logical_and(b == 0, fc == 0))
    def _():
        acc_ref[...] = jnp.zeros_like(acc_ref)

    @pl.when(valid_ref[b] == 1)
    def _():
        jbase = (b * BLK).astype(F32)

        @pl.when(fc == 0)
        def _():
            sub = jax.lax.broadcasted_iota(
                jnp.int32, (BLK, CTX), 0).astype(F32) + jbase
            dt = jnp.logical_or(p0t_ref[...] == sub, p1t_ref[...] == sub)
            zs_ref[...] = jnp.dot(dt.astype(BF16), z2_ref[...],
                                  preferred_element_type=F32).astype(BF16)
            eo_ref[...] = jnp.zeros_like(eo_ref)

        zsb = zs_ref[...]
        w1c = w1_ref[0].astype(BF16)
        w3c = w3_ref[0].astype(BF16)
        w2c = w2_ref[0].astype(BF16)
        h1 = jnp.dot(zsb, w1c, preferred_element_type=F32)
        h3 = jnp.dot(zsb, w3c, preferred_element_type=F32)
        hh = (h1 * jax.nn.sigmoid(h1) * h3).astype(BF16)
        eo_ref[...] += jnp.dot(hh, w2c, preferred_element_type=F32)

        @pl.when(fc == NFC - 1)
        def _():
            lane = jax.lax.broadcasted_iota(
                jnp.int32, (CTX, BLK), 1).astype(F32) + jbase
            comb = (g0_ref[...] * (p0_ref[...] == lane) +
                    g1_ref[...] * (p1_ref[...] == lane))
            acc_ref[...] += jnp.dot(comb.astype(BF16),
                                    eo_ref[...].astype(BF16),
                                    preferred_element_type=F32)

    @pl.when(jnp.logical_and(b == NBMAX - 1, fc == NFC - 1))
    def _():
        out_ref[...] = acc_ref[...] + h_ref[...]


def _moe(be, valid, z2b, p0t, p1t, p0, p1, g0, g1, hs, w1, w3, w2):
    grid_spec = pltpu.PrefetchScalarGridSpec(
        num_scalar_prefetch=2,
        grid=(NBMAX, NFC),
        in_specs=[
            pl.BlockSpec((CTX, HIDDEN), lambda b, f, be, vl: (0, 0)),
            pl.BlockSpec((1, CTX), lambda b, f, be, vl: (0, 0)),
            pl.BlockSpec((1, CTX), lambda b, f, be, vl: (0, 0)),
            pl.BlockSpec((CTX, 1), lambda b, f, be, vl: (0, 0)),
            pl.BlockSpec((CTX, 1), lambda b, f, be, vl: (0, 0)),
            pl.BlockSpec((CTX, 1), lambda b, f, be, vl: (0, 0)),
            pl.BlockSpec((CTX, 1), lambda b, f, be, vl: (0, 0)),
            pl.BlockSpec((CTX, HIDDEN), lambda b, f, be, vl: (0, 0)),
            pl.BlockSpec((1, HIDDEN, FC), lambda b, f, be, vl: (be[b], 0, f)),
            pl.BlockSpec((1, HIDDEN, FC), lambda b, f, be, vl: (be[b], 0, f)),
            pl.BlockSpec((1, FC, HIDDEN), lambda b, f, be, vl: (be[b], f, 0)),
        ],
        out_specs=pl.BlockSpec((CTX, HIDDEN), lambda b, f, be, vl: (0, 0)),
        scratch_shapes=[
            pltpu.VMEM((CTX, HIDDEN), F32),
            pltpu.VMEM((BLK, HIDDEN), BF16),
            pltpu.VMEM((BLK, HIDDEN), F32),
        ],
    )
    return pl.pallas_call(
        _k5_body,
        grid_spec=grid_spec,
        out_shape=jax.ShapeDtypeStruct((CTX, HIDDEN), F32),
    )(be, valid, z2b, p0t, p1t, p0, p1, g0, g1, hs, w1, w3, w2)


def kernel(x, g1, Wq, Wk, Wv, Wo, g2, Wr, W1, W2, W3):
    xs = x.reshape(CTX, HIDDEN)
    g1r = g1.reshape(1, HIDDEN)
    g2r = g2.reshape(1, HIDDEN)
    freqs = 1.0 / (10000.0 ** (jnp.arange(HALF, dtype=F32) / HALF))
    t = jnp.arange(CTX, dtype=F32)
    ang = t[:, None] * freqs[None, :]
    cos_t = jnp.cos(ang)
    sin_t = jnp.sin(ang)

    q, k, v = _qkv_rope(xs, g1r, Wq.astype(BF16), Wk.astype(BF16),
                        Wv.astype(BF16), cos_t, sin_t)
    o = _flash(q, k, v)
    hs, z2b, lg = _postattn(o, xs, Wo.astype(BF16), g2r, Wr)
    p0, p1, g0v, g1v, noff, aux = _route(lg)

    n = noff[0]
    off = noff[1]
    pad = jnp.floor((n + (BLK - 1)) / BLK) * BLK
    bidx = (jnp.arange(NBMAX, dtype=F32) * BLK)[:, None]
    inseg = (bidx >= off[None, :]) & (bidx < (off + pad)[None, :])
    be = jnp.sum(inseg * jnp.arange(E, dtype=F32)[None, :], axis=1)
    valid = bidx[:, 0] < jnp.sum(pad)
    be = jnp.where(valid, be, float(E - 1)).astype(jnp.int32)
    valid = valid.astype(jnp.int32)

    h2 = _moe(be, valid, z2b, p0.reshape(1, CTX), p1.reshape(1, CTX),
              p0, p1, g0v, g1v, hs,
              W1, W3, W2)
    return h2.reshape(1, CTX, HIDDEN), aux.reshape(())


# speedup vs baseline: 1.5446x; 1.3235x over previous
import functools
import math

import jax
import jax.numpy as jnp
from jax.experimental import pallas as pl
from jax.experimental.pallas import tpu as pltpu

HIDDEN = 768
FFN = 2048
HEADS = 12
DH = 64
HALF = 32
E = 8
CTX = 2048
BS = 256
QB = 512
BLK = 256
NBMAX = (CTX * 2 + E * (BLK - 1) + BLK - 1) // BLK
F32 = jnp.float32
BF16 = jnp.bfloat16


def _k1_body(x_ref, g1_ref, wq_ref, wk_ref, wv_ref, cos_ref, sin_ref,
             q_ref, k_ref, v_ref):
    xb = x_ref[...]
    z = xb * jax.lax.rsqrt(jnp.mean(xb * xb, axis=-1, keepdims=True) + 1e-5)
    z = (z * g1_ref[...]).astype(BF16)
    qf = jnp.dot(z, wq_ref[...], preferred_element_type=F32)
    kf = jnp.dot(z, wk_ref[...], preferred_element_type=F32)
    vf = jnp.dot(z, wv_ref[...], preferred_element_type=F32)
    c = cos_ref[...]
    s = sin_ref[...]
    for h in range(HEADS):
        a = slice(DH * h, DH * h + HALF)
        b = slice(DH * h + HALF, DH * h + DH)
        q1, q2 = qf[:, a], qf[:, b]
        q_ref[h, :, 0:HALF] = (q1 * c - q2 * s).astype(BF16)
        q_ref[h, :, HALF:DH] = (q1 * s + q2 * c).astype(BF16)
        k1_, k2_ = kf[:, a], kf[:, b]
        k_ref[h, :, 0:HALF] = (k1_ * c - k2_ * s).astype(BF16)
        k_ref[h, :, HALF:DH] = (k1_ * s + k2_ * c).astype(BF16)
        v_ref[h, :, :] = vf[:, DH * h:DH * h + DH].astype(BF16)


def _qkv_rope(xs, g1r, wq, wk, wv, cos_t, sin_t):
    shp = jax.ShapeDtypeStruct((HEADS, CTX, DH), BF16)
    return pl.pallas_call(
        _k1_body,
        grid=(CTX // BS,),
        in_specs=[
            pl.BlockSpec((BS, HIDDEN), lambda i: (i, 0)),
            pl.BlockSpec((1, HIDDEN), lambda i: (0, 0)),
            pl.BlockSpec((HIDDEN, HIDDEN), lambda i: (0, 0)),
            pl.BlockSpec((HIDDEN, HIDDEN), lambda i: (0, 0)),
            pl.BlockSpec((HIDDEN, HIDDEN), lambda i: (0, 0)),
            pl.BlockSpec((BS, HALF), lambda i: (i, 0)),
            pl.BlockSpec((BS, HALF), lambda i: (i, 0)),
        ],
        out_specs=[pl.BlockSpec((HEADS, BS, DH), lambda i: (0, i, 0))] * 3,
        out_shape=[shp, shp, shp],
    )(xs, g1r, wq, wk, wv, cos_t, sin_t)


def _k2_body(q_ref, k_ref, v_ref, o_ref):
    i = pl.program_id(1)
    q = q_ref[0]
    scale = 1.0 / math.sqrt(DH)
    rows = jax.lax.broadcasted_iota(jnp.int32, (QB, QB), 0) + i * QB

    def body(j, carry):
        m, l, acc = carry
        kb = k_ref[0, pl.ds(j * QB, QB), :]
        vb = v_ref[0, pl.ds(j * QB, QB), :]
        s = jax.lax.dot_general(q, kb, (((1,), (1,)), ((), ())),
                                preferred_element_type=F32) * scale
        cols = jax.lax.broadcasted_iota(jnp.int32, (QB, QB), 1) + j * QB
        s = jnp.where(rows >= cols, s, -1e30)
        m_new = jnp.maximum(m, jnp.max(s, axis=-1, keepdims=True))
        p = jnp.exp(s - m_new)
        corr = jnp.exp(m - m_new)
        l_new = l * corr + jnp.sum(p, axis=-1, keepdims=True)
        acc_new = acc * corr + jnp.dot(p.astype(BF16), vb,
                                       preferred_element_type=F32)
        return m_new, l_new, acc_new

    m0 = jnp.full((QB, 1), -1e30, F32)
    l0 = jnp.zeros((QB, 1), F32)
    a0 = jnp.zeros((QB, DH), F32)
    m, l, acc = jax.lax.fori_loop(0, i + 1, body, (m0, l0, a0))
    o_ref[0] = (acc / l).astype(BF16)


def _flash(q, k, v):
    return pl.pallas_call(
        _k2_body,
        grid=(HEADS, CTX // QB),
        in_specs=[
            pl.BlockSpec((1, QB, DH), lambda h, i: (h, i, 0)),
            pl.BlockSpec((1, CTX, DH), lambda h, i: (h, 0, 0)),
            pl.BlockSpec((1, CTX, DH), lambda h, i: (h, 0, 0)),
        ],
        out_specs=pl.BlockSpec((1, QB, DH), lambda h, i: (h, i, 0)),
        out_shape=jax.ShapeDtypeStruct((HEADS, CTX, DH), BF16),
    )(q, k, v)


def _k3_body(o_ref, x_ref, wo_ref, g2_ref, wr_ref, h_ref, z2_ref, lg_ref):
    oc = jnp.concatenate([o_ref[h] for h in range(HEADS)], axis=-1)
    attn = jnp.dot(oc, wo_ref[...], preferred_element_type=F32)
    hb = x_ref[...] + attn
    h_ref[...] = hb
    z2 = hb * jax.lax.rsqrt(jnp.mean(hb * hb, axis=-1, keepdims=True) + 1e-5)
    z2 = z2 * g2_ref[...]
    z2_ref[...] = z2.astype(BF16)
    lg_ref[...] = jnp.dot(z2, wr_ref[...], preferred_element_type=F32)


def _postattn(o, xs, wo, g2r, wr):
    return pl.pallas_call(
        _k3_body,
        grid=(CTX // BS,),
        in_specs=[
            pl.BlockSpec((HEADS, BS, DH), lambda i: (0, i, 0)),
            pl.BlockSpec((BS, HIDDEN), lambda i: (i, 0)),
            pl.BlockSpec((HIDDEN, HIDDEN), lambda i: (0, 0)),
            pl.BlockSpec((1, HIDDEN), lambda i: (0, 0)),
            pl.BlockSpec((HIDDEN, E), lambda i: (0, 0)),
        ],
        out_specs=[
            pl.BlockSpec((BS, HIDDEN), lambda i: (i, 0)),
            pl.BlockSpec((BS, HIDDEN), lambda i: (i, 0)),
            pl.BlockSpec((BS, E), lambda i: (i, 0)),
        ],
        out_shape=[
            jax.ShapeDtypeStruct((CTX, HIDDEN), F32),
            jax.ShapeDtypeStruct((CTX, HIDDEN), BF16),
            jax.ShapeDtypeStruct((CTX, E), F32),
        ],
    )(o, xs, wo, g2r, wr)


def _k4_body(lg_ref, pos0_ref, pos1_ref, g0_ref, g1_ref, noff_ref, aux_ref):
    lg = lg_ref[...]
    mx = jnp.max(lg, axis=-1, keepdims=True)
    ex = jnp.exp(lg - mx)
    probs = ex / jnp.sum(ex, axis=-1, keepdims=True)
    e_iota = jax.lax.broadcasted_iota(jnp.int32, (CTX, E), 1).astype(F32)
    m1 = jnp.max(probs, axis=-1, keepdims=True)
    i1 = jnp.min(jnp.where(probs == m1, e_iota, float(E)), axis=-1,
                 keepdims=True)
    probs_m = jnp.where(e_iota == i1, -1.0, probs)
    m2 = jnp.max(probs_m, axis=-1, keepdims=True)
    i2 = jnp.min(jnp.where(probs_m == m2, e_iota, float(E)), axis=-1,
                 keepdims=True)
    den = m1 + m2
    g0_ref[...] = m1 / den
    g1_ref[...] = m2 / den
    sel0 = (e_iota == i1)
    sel1 = (e_iota == i2)
    oh = sel0.astype(F32) + sel1.astype(F32)

    ri = jax.lax.broadcasted_iota(jnp.int32, (BS, BS), 0)
    ci = jax.lax.broadcasted_iota(jnp.int32, (BS, BS), 1)
    tri = (ci < ri).astype(BF16)
    carry = jnp.zeros((1, E), F32)
    parts = []
    for b in range(CTX // BS):
        seg = oh[b * BS:(b + 1) * BS, :]
        parts.append(jnp.dot(tri, seg.astype(BF16),
                             preferred_element_type=F32) + carry)
        carry = carry + jnp.sum(seg, axis=0, keepdims=True)
    c = jnp.concatenate(parts, axis=0)
    n = carry
    padded = jnp.floor((n + (BLK - 1)) / BLK) * BLK
    offs = []
    run = jnp.zeros((1, 1), F32)
    for e in range(E):
        offs.append(run)
        run = run + padded[:, e:e + 1]
    off = jnp.concatenate(offs, axis=1)
    c2 = c + off
    pos0_ref[...] = jnp.sum(jnp.where(sel0, c2, 0.0), axis=-1, keepdims=True)
    pos1_ref[...] = jnp.sum(jnp.where(sel1, c2, 0.0), axis=-1, keepdims=True)
    noff_ref[...] = jnp.concatenate([n, off], axis=0)
    f = jnp.mean(oh, axis=0, keepdims=True)
    pbar = jnp.mean(probs, axis=0, keepdims=True)
    aux_ref[...] = float(E) * jnp.sum(f * pbar, axis=-1, keepdims=True)


def _route(lg):
    return pl.pallas_call(
        _k4_body,
        out_shape=[
            jax.ShapeDtypeStruct((CTX, 1), F32),
            jax.ShapeDtypeStruct((CTX, 1), F32),
            jax.ShapeDtypeStruct((CTX, 1), F32),
            jax.ShapeDtypeStruct((CTX, 1), F32),
            jax.ShapeDtypeStruct((2, E), F32),
            jax.ShapeDtypeStruct((1, 1), F32),
        ],
    )(lg)


FC = FFN // 2
NFC = FFN // FC


def _k5_body(be_ref, valid_ref, z2_ref, p0t_ref, p1t_ref, p0_ref, p1_ref,
             g0_ref, g1_ref, h_ref, w1_ref, w3_ref, w2_ref, out_ref,
             acc_ref, zs_ref, eo_ref):
    b = pl.program_id(0)
    fc = pl.program_id(1)

    @pl.when(jnp.logical_and(b == 0, fc == 0))
    def _():
        acc_ref[...] = jnp.zeros_like(acc_ref)

    @pl.when(valid_ref[b] == 1)
    def _():
        jbase = (b * BLK).astype(F32)

        @pl.when(fc == 0)
        def _():
            sub = jax.lax.broadcasted_iota(
                jnp.int32, (BLK, CTX), 0).astype(F32) + jbase
            dt = jnp.logical_or(p0t_ref[...] == sub, p1t_ref[...] == sub)
            zs_ref[...] = jnp.dot(dt.astype(BF16), z2_ref[...],
                                  preferred_element_type=F32).astype(BF16)
            eo_ref[...] = jnp.zeros_like(eo_ref)

        zsb = zs_ref[...]
        w1c = w1_ref[0].astype(BF16)
        w3c = w3_ref[0].astype(BF16)
        w2c = w2_ref[0].astype(BF16)
        h1 = jnp.dot(zsb, w1c, preferred_element_type=F32)
        h3 = jnp.dot(zsb, w3c, preferred_element_type=F32)
        hh = (h1 * jax.nn.sigmoid(h1) * h3).astype(BF16)
        eo_ref[...] += jnp.dot(hh, w2c, preferred_element_type=F32)

        @pl.when(fc == NFC - 1)
        def _():
            lane = jax.lax.broadcasted_iota(
                jnp.int32, (CTX, BLK), 1).astype(F32) + jbase
            comb = (g0_ref[...] * (p0_ref[...] == lane) +
                    g1_ref[...] * (p1_ref[...] == lane))
            acc_ref[...] += jnp.dot(comb.astype(BF16),
                                    eo_ref[...].astype(BF16),
                                    preferred_element_type=F32)

    @pl.when(jnp.logical_and(b == NBMAX - 1, fc == NFC - 1))
    def _():
        out_ref[...] = acc_ref[...] + h_ref[...]


def _moe(be, valid, z2b, p0t, p1t, p0, p1, g0, g1, hs, w1, w3, w2):
    grid_spec = pltpu.PrefetchScalarGridSpec(
        num_scalar_prefetch=2,
        grid=(NBMAX, NFC),
        in_specs=[
            pl.BlockSpec((CTX, HIDDEN), lambda b, f, be, vl: (0, 0)),
            pl.BlockSpec((1, CTX), lambda b, f, be, vl: (0, 0)),
            pl.BlockSpec((1, CTX), lambda b, f, be, vl: (0, 0)),
            pl.BlockSpec((CTX, 1), lambda b, f, be, vl: (0, 0)),
            pl.BlockSpec((CTX, 1), lambda b, f, be, vl: (0, 0)),
            pl.BlockSpec((CTX, 1), lambda b, f, be, vl: (0, 0)),
            pl.BlockSpec((CTX, 1), lambda b, f, be, vl: (0, 0)),
            pl.BlockSpec((CTX, HIDDEN), lambda b, f, be, vl: (0, 0)),
            pl.BlockSpec((1, HIDDEN, FC), lambda b, f, be, vl: (be[b], 0, f)),
            pl.BlockSpec((1, HIDDEN, FC), lambda b, f, be, vl: (be[b], 0, f)),
            pl.BlockSpec((1, FC, HIDDEN), lambda b, f, be, vl: (be[b], f, 0)),
        ],
        out_specs=pl.BlockSpec((CTX, HIDDEN), lambda b, f, be, vl: (0, 0)),
        scratch_shapes=[
            pltpu.VMEM((CTX, HIDDEN), F32),
            pltpu.VMEM((BLK, HIDDEN), BF16),
            pltpu.VMEM((BLK, HIDDEN), F32),
        ],
    )
    return pl.pallas_call(
        _k5_body,
        grid_spec=grid_spec,
        out_shape=jax.ShapeDtypeStruct((CTX, HIDDEN), F32),
    )(be, valid, z2b, p0t, p1t, p0, p1, g0, g1, hs, w1, w3, w2)


def kernel(x, g1, Wq, Wk, Wv, Wo, g2, Wr, W1, W2, W3):
    xs = x.reshape(CTX, HIDDEN)
    g1r = g1.reshape(1, HIDDEN)
    g2r = g2.reshape(1, HIDDEN)
    freqs = 1.0 / (10000.0 ** (jnp.arange(HALF, dtype=F32) / HALF))
    t = jnp.arange(CTX, dtype=F32)
    ang = t[:, None] * freqs[None, :]
    cos_t = jnp.cos(ang)
    sin_t = jnp.sin(ang)

    q, k, v = _qkv_rope(xs, g1r, Wq.astype(BF16), Wk.astype(BF16),
                        Wv.astype(BF16), cos_t, sin_t)
    o = _flash(q, k, v)
    hs, z2b, lg = _postattn(o, xs, Wo.astype(BF16), g2r, Wr)
    p0, p1, g0v, g1v, noff, aux = _route(lg)

    n = noff[0]
    off = noff[1]
    pad = jnp.floor((n + (BLK - 1)) / BLK) * BLK
    bidx = (jnp.arange(NBMAX, dtype=F32) * BLK)[:, None]
    inseg = (bidx >= off[None, :]) & (bidx < (off + pad)[None, :])
    be = jnp.sum(inseg * jnp.arange(E, dtype=F32)[None, :], axis=1)
    valid = bidx[:, 0] < jnp.sum(pad)
    be = jnp.where(valid, be, float(E - 1)).astype(jnp.int32)
    valid = valid.astype(jnp.int32)

    h2 = _moe(be, valid, z2b, p0.reshape(1, CTX), p1.reshape(1, CTX),
              p0, p1, g0v, g1v, hs,
              W1, W3, W2)
    return h2.reshape(1, CTX, HIDDEN), aux.reshape(())
